# baseline (speedup 1.0000x reference)
import jax
import jax.numpy as jnp
from jax import lax
from jax.experimental import pallas as pl
from jax.experimental.pallas import tpu as pltpu
from jax.experimental.pallas import tpu_sc as plsc

_B = 4
_NG = 4096
_NS = (162, 642, 2562)
_ES = (480, 1920, 7680)
_NP8 = (168, 648, 2568)
_VC8 = (24, 84, 324)
_EC = (60, 240, 960)
_NVG = (2, 6, 21)
_NEG = (4, 15, 60)
_SD = (512, 1952, 7712)
_G = 512
_NB = _NG // _G

_W_CHAMFER_OPP = 0.55
_W_LAPLACE = 0.5
_W_MOVE = 0.1
_W_EDGE = 0.1
_W_NORMAL = 0.00016
_W_RECONST = 0.1
_LAP_CONST = (0.2, 1.0, 1.0)


def _chamfer_body(gt_ref, p0_ref, p1_ref, p2_ref, gi_ref, rc_ref, gd_ref,
                  pd_ref, mk_ref, sums_ref, k0_ref, k1_ref, k2_ref):
    nb = pl.program_id(1)
    first = nb == 0
    last = nb == _NB - 1
    gt = gt_ref[0]
    gtm2 = gt * -2.0
    gg = jnp.sum(gt * gt, axis=1, keepdims=True)
    lane16 = lax.broadcasted_iota(jnp.int32, (1, 16), 1)
    sv = jnp.zeros((1, 16), jnp.float32)
    p_refs = (p0_ref, p1_ref, p2_ref)
    k_refs = (k0_ref, k1_ref, k2_ref)
    for l in range(3):
        np_ = _NP8[l]
        pr = p_refs[l][0]
        pp = jnp.sum(pr * pr, axis=1)[None, :]
        m2 = lax.dot_general(gtm2, pr, (((1,), (1,)), ((), ())),
                             preferred_element_type=jnp.float32)
        d = jnp.maximum((gg + pp) + m2, 0.0)

        rows = lax.broadcasted_iota(jnp.int32, (_G, np_), 0) + nb * _G
        key = ((lax.bitcast_convert_type(d, jnp.int32) & jnp.int32(~0xFFF))
               | rows)
        bkey = jnp.min(key, axis=0, keepdims=True)
        prev = jnp.where(first, jnp.int32(0x7F7FFFFF), k_refs[l][0])
        newk = jnp.minimum(bkey, prev)
        k_refs[l][0] = newk

        s1 = jnp.sum(jnp.min(d, axis=1, keepdims=True))
        sv = sv + jnp.where(lane16 == l, s1, 0.0)

        @pl.when(last)
        def _():
            io = lax.broadcasted_iota(jnp.int32, (1, np_), 1)
            dmin = lax.bitcast_convert_type(newk & jnp.int32(~0xFFF),
                                            jnp.float32)
            s2 = jnp.sum(jnp.where(io < _NS[l], dmin, 0.0))
            sums_ref[0] = sums_ref[0] + jnp.where(lane16 == 3 + l, s2, 0.0)

    prev_s = jnp.where(first, 0.0, sums_ref[0])
    sums_ref[0] = prev_s + sv

    @pl.when(first & (pl.program_id(0) == 0))
    def _():
        p = jnp.clip(rc_ref[...], 1e-7, 1.0 - 1e-7)
        gi = gi_ref[...]
        bce = jnp.sum(gi * jnp.log(p) + (1.0 - gi) * jnp.log(1.0 - p))
        m = (mk_ref[...] > 0.5).astype(jnp.float32)
        dlt = pd_ref[...] - gd_ref[...]
        ad = jnp.abs(dlt)
        sl1 = jnp.where(ad < 1.0, 0.5 * dlt * dlt, ad - 0.5)
        dnum = jnp.sum(sl1 * m)
        dden = jnp.sum(m)
        sums_ref[0] = (sums_ref[0] + jnp.where(lane16 == 6, bce, 0.0)
                       + jnp.where(lane16 == 7, dnum, 0.0)
                       + jnp.where(lane16 == 8, dden, 0.0))


def _chamfer_call(gt_points, pc8, gi, rc, gd, pd, mk):
    outs = pl.pallas_call(
        _chamfer_body,
        grid=(_B, _NB),
        in_specs=[
            pl.BlockSpec((1, _G, 3), lambda b, nb: (b, nb, 0)),
            pl.BlockSpec((1, _NP8[0], 3), lambda b, nb: (b, 0, 0)),
            pl.BlockSpec((1, _NP8[1], 3), lambda b, nb: (b, 0, 0)),
            pl.BlockSpec((1, _NP8[2], 3), lambda b, nb: (b, 0, 0)),
            pl.BlockSpec((12, 50176), lambda b, nb: (0, 0)),
            pl.BlockSpec((12, 50176), lambda b, nb: (0, 0)),
            pl.BlockSpec((4, 50176), lambda b, nb: (0, 0)),
            pl.BlockSpec((4, 50176), lambda b, nb: (0, 0)),
            pl.BlockSpec((4, 50176), lambda b, nb: (0, 0)),
        ],
        out_specs=[pl.BlockSpec((1, 1, 16), lambda b, nb: (b, 0, 0))]
        + [pl.BlockSpec((1, 1, _NP8[l]), lambda b, nb: (b, 0, 0))
           for l in range(3)],
        out_shape=[jax.ShapeDtypeStruct((_B, 1, 16), jnp.float32)]
        + [jax.ShapeDtypeStruct((_B, 1, _NP8[l]), jnp.int32)
           for l in range(3)],
    )(gt_points, pc8[0], pc8[1], pc8[2], gi, rc, gd, pd, mk)
    return outs[0], outs[1:4]


def _rsqrt16(x):
    i = plsc.bitcast(x, jnp.int32)
    i = jnp.int32(0x5F3759DF) - (i >> 1)
    y = plsc.bitcast(i, jnp.float32)
    for _ in range(3):
        y = y * (1.5 - 0.5 * x * y * y)
    return y


def _sc_body(*refs):
    pc_h = refs[0:3]
    pb_h = refs[3:6]
    nrm_h = refs[6]
    i2_h = refs[7:10]
    lap_h = refs[10:13]
    edg_h = refs[13:16]
    out_h = refs[16]
    sc = refs[17:]
    crd_t = sc[0:3]
    dcr_t = sc[3:6]
    i2_t = sc[6:9]
    lap_t = sc[9:12]
    edg_t = sc[12:15]
    nrm_t = sc[15]
    out_t = sc[16]
    sem = sc[17]

    wid = lax.axis_index("s") * 2 + lax.axis_index("c")
    b = wid // 8
    ch = wid % 8

    cps = [pltpu.async_copy(nrm_h.at[pl.ds(b * (_NG * 3), _NG * 3)],
                            nrm_t, sem)]
    for lvl in range(3):
        np3 = _NP8[lvl] * 3
        vc, ec = _VC8[lvl], _EC[lvl]
        cps.append(pltpu.async_copy(
            pc_h[lvl].at[pl.ds(b * np3, np3)],
            crd_t[lvl].at[pl.ds(0, np3)], sem))
        cps.append(pltpu.async_copy(
            pb_h[lvl].at[pl.ds(b * np3, np3)],
            dcr_t[lvl].at[pl.ds(0, np3)], sem))
        cps.append(pltpu.async_copy(
            i2_h[lvl].at[pl.ds(b * _NP8[lvl], _NP8[lvl])], i2_t[lvl], sem))
        cps.append(pltpu.async_copy(
            lap_h[lvl].at[pl.ds(ch * (vc * 10), vc * 10)], lap_t[lvl], sem))
        cps.append(pltpu.async_copy(
            edg_h[lvl].at[pl.ds(ch * (ec * 2), ec * 2)], edg_t[lvl], sem))
    for c in cps:
        c.wait()

    zero16 = jnp.zeros((16,), jnp.float32)
    iota16 = lax.iota(jnp.int32, 16)
    for lvl in range(3):
        n = _NS[lvl]
        vc, ec = _VC8[lvl], _EC[lvl]
        crd, dcr = crd_t[lvl], dcr_t[lvl]
        i2v_t, lap, edg = i2_t[lvl], lap_t[lvl], edg_t[lvl]

        def dbody(g, c):
            off = g * 16
            dcr[pl.ds(off, 16)] = dcr[pl.ds(off, 16)] - crd[pl.ds(off, 16)]
            return c
        lax.fori_loop(0, _SD[lvl] // 16, dbody, 0)

        def vbody(g, carry):
            lap_a, mv_a = carry
            vi = g * 16 + iota16
            vc16 = jnp.minimum(vi, vc - 1)
            b10 = vc16 * 10
            gv = ch * vc + vc16
            wv = jnp.where((vi < vc) & (gv < n), 1.0, 0.0)
            g3 = jnp.minimum(gv, n - 1) * 3
            sx = plsc.load_gather(dcr, [g3])
            sy = plsc.load_gather(dcr, [g3 + 1])
            sz = plsc.load_gather(dcr, [g3 + 2])
            cnt = plsc.load_gather(lap, [b10 + 9])
            rcv = 1.0 / cnt.astype(jnp.float32)
            ax = zero16
            ay = zero16
            az = zero16
            for k in range(8):
                nk = plsc.load_gather(lap, [b10 + k])
                wk = jnp.where(nk >= 0, 1.0, 0.0)
                n3 = jnp.maximum(nk, 0) * 3
                ax = ax + wk * plsc.load_gather(dcr, [n3])
                ay = ay + wk * plsc.load_gather(dcr, [n3 + 1])
                az = az + wk * plsc.load_gather(dcr, [n3 + 2])
            lx = (sx - ax * rcv) * wv
            ly = (sy - ay * rcv) * wv
            lz = (sz - az * rcv) * wv
            lap_a = lap_a + lx * lx + ly * ly + lz * lz
            mv_a = mv_a + (sx * sx + sy * sy + sz * sz) * wv
            return (lap_a, mv_a)
        lap_v, mv_v = lax.fori_loop(0, _NVG[lvl], vbody, (zero16, zero16))

        def ebody(g, carry):
            eg_a, nr_a = carry
            ei = g * 16 + iota16
            m = jnp.where(ei < ec, 1.0, 0.0)
            e2 = jnp.minimum(ei, ec - 1) * 2
            a0 = plsc.load_gather(edg, [e2])
            a1 = plsc.load_gather(edg, [e2 + 1])
            a03 = a0 * 3
            a13 = a1 * 3
            dex = plsc.load_gather(crd, [a03]) - plsc.load_gather(crd, [a13])
            dey = (plsc.load_gather(crd, [a03 + 1])
                   - plsc.load_gather(crd, [a13 + 1]))
            dez = (plsc.load_gather(crd, [a03 + 2])
                   - plsc.load_gather(crd, [a13 + 2]))
            se = dex * dex + dey * dey + dez * dez
            i2v = plsc.load_gather(i2v_t, [a0]) & jnp.int32(0xFFF)
            i3 = i2v * 3
            nxv = plsc.load_gather(nrm_t, [i3])
            nyv = plsc.load_gather(nrm_t, [i3 + 1])
            nzv = plsc.load_gather(nrm_t, [i3 + 2])
            dp = dex * nxv + dey * nyv + dez * nzv
            sn = nxv * nxv + nyv * nyv + nzv * nzv
            rse = _rsqrt16(jnp.maximum(se, 1e-24))
            rsn = _rsqrt16(jnp.maximum(sn, 1e-24))
            return (eg_a + se * m, nr_a + jnp.abs(dp) * rse * rsn * m)
        eg_v, nr_v = lax.fori_loop(0, _NEG[lvl], ebody, (zero16, zero16))

        out_t[pl.ds((4 * lvl + 0) * 16, 16)] = lap_v
        out_t[pl.ds((4 * lvl + 1) * 16, 16)] = mv_v
        out_t[pl.ds((4 * lvl + 2) * 16, 16)] = eg_v
        out_t[pl.ds((4 * lvl + 3) * 16, 16)] = nr_v
    for r in range(12, 16):
        out_t[pl.ds(r * 16, 16)] = zero16
    pltpu.sync_copy(out_t, out_h.at[pl.ds(wid * 256, 256)])


def _sc_call(args):
    mesh = plsc.VectorSubcoreMesh(core_axis_name="c", subcore_axis_name="s")
    f = pl.kernel(
        _sc_body,
        out_type=jax.ShapeDtypeStruct((8192,), jnp.float32),
        mesh=mesh,
        compiler_params=pltpu.CompilerParams(needs_layout_passes=False),
        scratch_types=(
            [pltpu.VMEM((_SD[lvl],), jnp.float32) for lvl in range(3)]
            + [pltpu.VMEM((_SD[lvl],), jnp.float32) for lvl in range(3)]
            + [pltpu.VMEM((_NP8[lvl],), jnp.int32) for lvl in range(3)]
            + [pltpu.VMEM((_VC8[lvl] * 10,), jnp.int32) for lvl in range(3)]
            + [pltpu.VMEM((_EC[lvl] * 2,), jnp.int32) for lvl in range(3)]
            + [pltpu.VMEM((_NG * 3,), jnp.float32),
               pltpu.VMEM((256,), jnp.float32),
               pltpu.SemaphoreType.DMA]
        ),
    )
    return f(*args)


@jax.jit
def kernel(gt_points, gt_normals, gt_images, gt_depth, mask, pred_depth,
           reconst, pred_coord_0, pred_coord_1, pred_coord_2,
           pred_before_0, pred_before_1, pred_before_2,
           lap_idx_0, lap_idx_1, lap_idx_2, edges_0, edges_1, edges_2):
    pcs = (pred_coord_0, pred_coord_1, pred_coord_2)
    pbs = (pred_before_0, pred_before_1, pred_before_2)
    laps = (lap_idx_0, lap_idx_1, lap_idx_2)
    edgs = (edges_0, edges_1, edges_2)
    pc8 = [jnp.pad(p, ((0, 0), (0, _NP8[i] - _NS[i]), (0, 0)),
                   constant_values=1e9) for i, p in enumerate(pcs)]
    pb8 = [jnp.pad(p, ((0, 0), (0, _NP8[i] - _NS[i]), (0, 0)),
                   constant_values=1e9) for i, p in enumerate(pbs)]

    sums, i2s = _chamfer_call(
        gt_points, pc8,
        gt_images.reshape(12, 50176), reconst.reshape(12, 50176),
        gt_depth.reshape(4, 50176), pred_depth.reshape(4, 50176),
        mask.reshape(4, 50176))
    sums = sums.reshape(_B, 16)

    sc_args = ([p.reshape(-1) for p in pc8] + [p.reshape(-1) for p in pb8]
               + [gt_normals.reshape(-1)]
               + [i2.reshape(-1) for i2 in i2s]
               + [jnp.pad(laps[l], ((0, 8 * _VC8[l] - _NS[l]), (0, 0)),
                          constant_values=-1).reshape(-1) for l in range(3)]
               + [e.reshape(-1) for e in edgs])
    sc_out = _sc_call(sc_args).reshape(32, 16, 16)
    q = jnp.sum(sc_out, axis=(0, 2))

    chamfer_loss = 0.0
    lap_loss = 0.0
    move_loss = 0.0
    edge_loss = 0.0
    normal_loss = 0.0
    for l in range(3):
        n = jnp.float32(_NS[l])
        e = jnp.float32(_ES[l])
        chamfer_loss = chamfer_loss + (jnp.sum(sums[:, l]) / _NG
                                       + _W_CHAMFER_OPP * jnp.sum(sums[:, 3 + l]) / n)
        lap_loss = lap_loss + _LAP_CONST[l] * q[4 * l + 0] / n
        if l > 0:
            move_loss = move_loss + _LAP_CONST[l] * q[4 * l + 1] / n
        edge_loss = edge_loss + q[4 * l + 2] / e
        normal_loss = normal_loss + q[4 * l + 3] / e
    image_loss = -sums[0, 6] / jnp.float32(12 * 50176)
    depth_loss = sums[0, 7] / jnp.maximum(sums[0, 8], 1.0)
    loss = (chamfer_loss + image_loss * _W_RECONST + _W_LAPLACE * lap_loss
            + _W_MOVE * move_loss + _W_EDGE * edge_loss
            + _W_NORMAL * normal_loss + depth_loss)
    return loss
